# baseline (speedup 1.0000x reference)
import functools

import jax
import jax.numpy as jnp
from jax import lax
from jax.experimental import pallas as pl
from jax.experimental.pallas import tpu as pltpu
from jax.experimental.pallas import tpu_sc as plsc

N = 10000
E = 320000
D = 128

NC = 2
NS = 16
NW = NC * NS
EPT = E // NW
CHUNK = 80
NCHP = 128
EPTP = NCHP * CHUNK
NSINK = 8
NP = N + NSINK
KR = 4
KI = 8
SLAB = 624
REM = N - NS * SLAB
ZREM = NP - NS * SLAB
ZR = 24


def _segsum_body(g_hbm, srcp_hbm, dstp_hbm, out_hbm, agg_sh, zbuf,
                 *bufs_and_sems):
    rows = bufs_and_sems[0:KR]
    sidxb = bufs_and_sems[KR:KR + KI]
    didxb = bufs_and_sems[KR + KI:KR + 2 * KI]
    o = KR + 2 * KI
    gsem = bufs_and_sems[o:o + KR]
    ssem = bufs_and_sems[o + KR:o + 2 * KR]
    o += 2 * KR
    issem = bufs_and_sems[o:o + KI]
    idsem = bufs_and_sems[o + KI:o + 2 * KI]

    c = lax.axis_index("c")
    s = lax.axis_index("s")
    wid = c * NS + s
    base = wid * EPTP

    zero16 = jnp.zeros((16,), jnp.float32)

    def zfill(i, carry):
        for k in range(D // 16):
            zbuf[i, pl.ds(k * 16, 16)] = zero16
        return carry

    lax.fori_loop(0, ZR, zfill, 0)
    for q in range(SLAB // ZR):
        pltpu.sync_copy(zbuf, agg_sh.at[pl.ds(s * SLAB + q * ZR, ZR)])

    @pl.when(s == NS - 1)
    def _zero_rem():
        pltpu.sync_copy(zbuf, agg_sh.at[pl.ds(NS * SLAB, ZREM)])

    plsc.subcore_barrier()

    def fire_idx(j, b8):
        pltpu.async_copy(srcp_hbm.at[pl.ds(base + j * CHUNK, CHUNK)],
                         sidxb[b8], issem[b8])
        pltpu.async_copy(dstp_hbm.at[pl.ds(base + j * CHUNK, CHUNK)],
                         didxb[b8], idsem[b8])

    def wait_idx(j, b8):
        pltpu.make_async_copy(srcp_hbm.at[pl.ds(base + j * CHUNK, CHUNK)],
                              sidxb[b8], issem[b8]).wait()
        pltpu.make_async_copy(dstp_hbm.at[pl.ds(base + j * CHUNK, CHUNK)],
                              didxb[b8], idsem[b8]).wait()

    def fire_gather(b4, b8):
        pltpu.async_copy(g_hbm.at[sidxb[b8]], rows[b4], gsem[b4])

    def wait_gather(b4, b8):
        pltpu.make_async_copy(g_hbm.at[sidxb[b8]], rows[b4], gsem[b4]).wait()

    def fire_scatter(b4, b8):
        pltpu.async_copy(rows[b4], agg_sh.at[didxb[b8]], ssem[b4], add=True)

    def wait_scatter(b4, b8):
        pltpu.make_async_copy(rows[b4], agg_sh.at[didxb[b8]],
                              ssem[b4]).wait()

    for j in range(KI):
        fire_idx(j, j)
    for j in range(2):
        wait_idx(j, j)
        fire_gather(j, j)
    for j in range(2):
        wait_gather(j % 4, j % 8)
        fire_scatter(j % 4, j % 8)
        wait_idx(j + 2, (j + 2) % 8)
        fire_gather((j + 2) % 4, (j + 2) % 8)

    def steady(g, carry):
        jb = 8 * g + 2
        for k in range(8):
            j = jb + k
            b4 = (2 + k) % 4
            b8 = (2 + k) % 8
            wait_gather(b4, b8)
            fire_scatter(b4, b8)
            wait_scatter((k % 4), (k % 8))
            wait_idx(j + 2, (4 + k) % 8)
            fire_gather((4 + k) % 4, (4 + k) % 8)
            fire_idx(j + 6, k % 8)
        return carry

    lax.fori_loop(0, 15, steady, 0)

    for j in range(122, 126):
        wait_gather(j % 4, j % 8)
        fire_scatter(j % 4, j % 8)
        wait_scatter((j - 2) % 4, (j - 2) % 8)
        wait_idx(j + 2, (j + 2) % 8)
        fire_gather((j + 2) % 4, (j + 2) % 8)
    for j in range(126, 128):
        wait_gather(j % 4, j % 8)
        fire_scatter(j % 4, j % 8)
        wait_scatter((j - 2) % 4, (j - 2) % 8)
    wait_scatter(126 % 4, 126 % 8)
    wait_scatter(127 % 4, 127 % 8)
    plsc.subcore_barrier()

    pltpu.sync_copy(agg_sh.at[pl.ds(s * SLAB, SLAB)],
                    out_hbm.at[c, pl.ds(s * SLAB, SLAB)])

    @pl.when(s == NS - 1)
    def _drain_rem():
        pltpu.sync_copy(agg_sh.at[pl.ds(NS * SLAB, REM)],
                        out_hbm.at[c, pl.ds(NS * SLAB, REM)])


def _make_segsum():
    mesh = plsc.VectorSubcoreMesh(core_axis_name="c", subcore_axis_name="s")
    scratch = [
        pltpu.VMEM_SHARED((NP, D), jnp.float32),
        pltpu.VMEM((ZR, D), jnp.float32),
    ]
    scratch += [pltpu.VMEM((CHUNK, D), jnp.float32) for _ in range(KR)]
    scratch += [pltpu.VMEM((CHUNK,), jnp.int32) for _ in range(2 * KI)]
    scratch += [pltpu.SemaphoreType.DMA for _ in range(2 * KR + 2 * KI)]
    return pl.kernel(
        _segsum_body,
        out_type=jax.ShapeDtypeStruct((NC, N, D), jnp.float32),
        mesh=mesh,
        scratch_types=scratch,
    )


def _mm_body(x_ref, w_ref, o_ref):
    o_ref[...] = lax.dot_general(
        x_ref[...], w_ref[...], (((1,), (1,)), ((), ())),
        preferred_element_type=jnp.float32)


def _mm(x, w):
    return pl.pallas_call(
        _mm_body,
        grid=(10,),
        in_specs=[
            pl.BlockSpec((N // 10, D), lambda i: (i, 0)),
            pl.BlockSpec((D, D), lambda i: (0, 0)),
        ],
        out_specs=pl.BlockSpec((N // 10, D), lambda i: (i, 0)),
        out_shape=jax.ShapeDtypeStruct((N, D), jnp.float32),
    )(x, w)


def _combine_mm_body(scale_ref, g_ref, agg_ref, b_ref, w_ref, o_ref):
    z = (scale_ref[0] * g_ref[...] + agg_ref[0] + agg_ref[1]
         + b_ref[...][None, :])
    o_ref[...] = lax.dot_general(
        z, w_ref[...], (((1,), (1,)), ((), ())),
        preferred_element_type=jnp.float32)


def _combine_mm(scale, g, agg, b, w):
    return pl.pallas_call(
        _combine_mm_body,
        grid=(10,),
        in_specs=[
            pl.BlockSpec(memory_space=pltpu.SMEM),
            pl.BlockSpec((N // 10, D), lambda i: (i, 0)),
            pl.BlockSpec((NC, N // 10, D), lambda i: (0, i, 0)),
            pl.BlockSpec((D,), lambda i: (0,)),
            pl.BlockSpec((D, D), lambda i: (0, 0)),
        ],
        out_specs=pl.BlockSpec((N // 10, D), lambda i: (i, 0)),
        out_shape=jax.ShapeDtypeStruct((N, D), jnp.float32),
    )(scale, g, agg, b, w)


def _combine_body(scale_ref, g_ref, agg_ref, b_ref, o_ref):
    o_ref[...] = (scale_ref[0] * g_ref[...] + agg_ref[0] + agg_ref[1]
                  + b_ref[...][None, :])


def _combine(scale, g, agg, b):
    return pl.pallas_call(
        _combine_body,
        grid=(10,),
        in_specs=[
            pl.BlockSpec(memory_space=pltpu.SMEM),
            pl.BlockSpec((N // 10, D), lambda i: (i, 0)),
            pl.BlockSpec((NC, N // 10, D), lambda i: (0, i, 0)),
            pl.BlockSpec((D,), lambda i: (0,)),
        ],
        out_specs=pl.BlockSpec((N // 10, D), lambda i: (i, 0)),
        out_shape=jax.ShapeDtypeStruct((N, D), jnp.float32),
    )(scale, g, agg, b)


_segsum = _make_segsum()


def kernel(feats, edge_index, W1, b1, W2, b2, eps1, eps2):
    npad = EPTP - EPT
    src = edge_index[0].reshape(NW, EPT)
    dst = edge_index[1].reshape(NW, EPT)
    srcp = jnp.pad(src, ((0, 0), (0, npad))).reshape(-1)
    sink = jnp.broadcast_to(
        N + (jnp.arange(npad, dtype=jnp.int32) % NSINK), (NW, npad))
    dstp = jnp.concatenate([dst, sink], axis=1).reshape(-1)
    scale1 = (1.0 + eps1).reshape(1)
    scale2 = (1.0 + eps2).reshape(1)
    g1 = _mm(feats, W1)
    agg1 = _segsum(g1, srcp, dstp)
    g2 = _combine_mm(scale1, g1, agg1, b1, W2)
    agg2 = _segsum(g2, srcp, dstp)
    return _combine(scale2, g2, agg2, b2)


# speedup vs baseline: 2.3498x; 2.3498x over previous
import functools

import jax
import jax.numpy as jnp
from jax import lax
from jax.experimental import pallas as pl
from jax.experimental.pallas import tpu as pltpu
from jax.experimental.pallas import tpu_sc as plsc

N = 10000
E = 320000
D = 128

NC = 2
NS = 16
NW = NC * NS
EPT = E // NW
CHUNK = 80
NCH = EPT // CHUNK
SLAB = 624
REM = N - NS * SLAB
ZR = 16


def _segsum_body(g_hbm, srcf_hbm, dstf_hbm, out_hbm, agg_sh, sidx, didx,
                 zbuf, rows0, rows1, gsem0, gsem1, ssem0, ssem1):
    rows = (rows0, rows1)
    gsems = (gsem0, gsem1)
    ssems = (ssem0, ssem1)
    c = lax.axis_index("c")
    s = lax.axis_index("s")
    wid = c * NS + s

    zero16 = jnp.zeros((16,), jnp.float32)

    def zfill(i, carry):
        for k in range(D // 16):
            zbuf[i, pl.ds(k * 16, 16)] = zero16
        return carry

    lax.fori_loop(0, ZR, zfill, 0)
    for q in range(SLAB // ZR):
        pltpu.sync_copy(zbuf, agg_sh.at[pl.ds(s * SLAB + q * ZR, ZR)])

    @pl.when(s == NS - 1)
    def _zero_rem():
        pltpu.sync_copy(zbuf, agg_sh.at[pl.ds(NS * SLAB, REM)])

    pltpu.sync_copy(srcf_hbm.at[pl.ds(wid * EPT, EPT)], sidx)
    pltpu.sync_copy(dstf_hbm.at[pl.ds(wid * EPT, EPT)], didx)
    plsc.subcore_barrier()

    def fire_gather(j, b):
        pltpu.async_copy(g_hbm.at[sidx.at[pl.ds(j * CHUNK, CHUNK)]],
                         rows[b], gsems[b])

    def wait_gather(j, b):
        pltpu.make_async_copy(g_hbm.at[sidx.at[pl.ds(j * CHUNK, CHUNK)]],
                              rows[b], gsems[b]).wait()

    def fire_scatter(j, b):
        pltpu.async_copy(rows[b], agg_sh.at[didx.at[pl.ds(j * CHUNK, CHUNK)]],
                         ssems[b], add=True)

    def wait_scatter(j, b):
        pltpu.make_async_copy(rows[b],
                              agg_sh.at[didx.at[pl.ds(j * CHUNK, CHUNK)]],
                              ssems[b]).wait()

    fire_gather(0, 0)
    fire_gather(1, 1)

    def group(g, carry):
        j0 = 2 * g
        for b in range(2):
            wait_gather(j0 + b, b)
            fire_scatter(j0 + b, b)
        for b in range(2):
            wait_scatter(j0 + b, b)
            fire_gather(j0 + 2 + b, b)
        return carry

    lax.fori_loop(0, (NCH - 3) // 2, group, 0)
    wait_gather(NCH - 3, 0)
    fire_scatter(NCH - 3, 0)
    wait_gather(NCH - 2, 1)
    fire_scatter(NCH - 2, 1)
    wait_scatter(NCH - 3, 0)
    fire_gather(NCH - 1, 0)
    wait_gather(NCH - 1, 0)
    fire_scatter(NCH - 1, 0)
    wait_scatter(NCH - 2, 1)
    wait_scatter(NCH - 1, 0)
    plsc.subcore_barrier()

    pltpu.sync_copy(agg_sh.at[pl.ds(s * SLAB, SLAB)],
                    out_hbm.at[c, pl.ds(s * SLAB, SLAB)])

    @pl.when(s == NS - 1)
    def _drain_rem():
        pltpu.sync_copy(agg_sh.at[pl.ds(NS * SLAB, REM)],
                        out_hbm.at[c, pl.ds(NS * SLAB, REM)])


def _make_segsum():
    mesh = plsc.VectorSubcoreMesh(core_axis_name="c", subcore_axis_name="s")
    scratch = [
        pltpu.VMEM_SHARED((N, D), jnp.float32),
        pltpu.VMEM((EPT,), jnp.int32),
        pltpu.VMEM((EPT,), jnp.int32),
        pltpu.VMEM((ZR, D), jnp.float32),
        pltpu.VMEM((CHUNK, D), jnp.float32),
        pltpu.VMEM((CHUNK, D), jnp.float32),
        pltpu.SemaphoreType.DMA,
        pltpu.SemaphoreType.DMA,
        pltpu.SemaphoreType.DMA,
        pltpu.SemaphoreType.DMA,
    ]
    return pl.kernel(
        _segsum_body,
        out_type=jax.ShapeDtypeStruct((NC, N, D), jnp.float32),
        mesh=mesh,
        scratch_types=scratch,
    )


def _mm_body(x_ref, w_ref, o_ref):
    o_ref[...] = lax.dot_general(
        x_ref[...], w_ref[...], (((1,), (1,)), ((), ())),
        preferred_element_type=jnp.float32)


def _mm(x, w):
    return pl.pallas_call(
        _mm_body,
        grid=(10,),
        in_specs=[
            pl.BlockSpec((N // 10, D), lambda i: (i, 0)),
            pl.BlockSpec((D, D), lambda i: (0, 0)),
        ],
        out_specs=pl.BlockSpec((N // 10, D), lambda i: (i, 0)),
        out_shape=jax.ShapeDtypeStruct((N, D), jnp.float32),
    )(x, w)


def _combine_mm_body(scale_ref, g_ref, agg_ref, b_ref, w_ref, o_ref):
    z = (scale_ref[0] * g_ref[...] + agg_ref[0] + agg_ref[1]
         + b_ref[...][None, :])
    o_ref[...] = lax.dot_general(
        z, w_ref[...], (((1,), (1,)), ((), ())),
        preferred_element_type=jnp.float32)


def _combine_mm(scale, g, agg, b, w):
    return pl.pallas_call(
        _combine_mm_body,
        grid=(10,),
        in_specs=[
            pl.BlockSpec(memory_space=pltpu.SMEM),
            pl.BlockSpec((N // 10, D), lambda i: (i, 0)),
            pl.BlockSpec((NC, N // 10, D), lambda i: (0, i, 0)),
            pl.BlockSpec((D,), lambda i: (0,)),
            pl.BlockSpec((D, D), lambda i: (0, 0)),
        ],
        out_specs=pl.BlockSpec((N // 10, D), lambda i: (i, 0)),
        out_shape=jax.ShapeDtypeStruct((N, D), jnp.float32),
    )(scale, g, agg, b, w)


def _combine_body(scale_ref, g_ref, agg_ref, b_ref, o_ref):
    o_ref[...] = (scale_ref[0] * g_ref[...] + agg_ref[0] + agg_ref[1]
                  + b_ref[...][None, :])


def _combine(scale, g, agg, b):
    return pl.pallas_call(
        _combine_body,
        grid=(10,),
        in_specs=[
            pl.BlockSpec(memory_space=pltpu.SMEM),
            pl.BlockSpec((N // 10, D), lambda i: (i, 0)),
            pl.BlockSpec((NC, N // 10, D), lambda i: (0, i, 0)),
            pl.BlockSpec((D,), lambda i: (0,)),
        ],
        out_specs=pl.BlockSpec((N // 10, D), lambda i: (i, 0)),
        out_shape=jax.ShapeDtypeStruct((N, D), jnp.float32),
    )(scale, g, agg, b)


_segsum = _make_segsum()


def kernel(feats, edge_index, W1, b1, W2, b2, eps1, eps2):
    srcf = edge_index[0]
    dstf = edge_index[1]
    scale1 = (1.0 + eps1).reshape(1)
    scale2 = (1.0 + eps2).reshape(1)
    g1 = _mm(feats, W1)
    agg1 = _segsum(g1, srcf, dstf)
    g2 = _combine_mm(scale1, g1, agg1, b1, W2)
    agg2 = _segsum(g2, srcf, dstf)
    return _combine(scale2, g2, agg2, b2)
